# initial kernel scaffold (unmeasured)
import jax
import jax.numpy as jnp
from jax import lax
from jax.experimental import pallas as pl
from jax.experimental.pallas import tpu as pltpu


def kernel(Q, K, V, bt, lens):
    B, _, H, D = Q.shape
    P_loc, BS = K.shape[0], K.shape[1]
    NB = bt.shape[1]
    KT = P_loc * BS
    scale = D ** -0.5

    Qs = Q.reshape(B, H, D)
    Kf = K.reshape(KT, H, D)
    Vf = V.reshape(KT, H, D)
    lens2 = lens.reshape(B, 1)

    def body(q_ref, k_ref, v_ref, bt_ref, lens_ref, out_ref,
             o_send, o_recv, l_send, l_recv, send_sems, recv_sems):
        my_x = lax.axis_index("x")
        my_y = lax.axis_index("y")
        peer = (my_x, 1 - my_y)

        barrier = pltpu.get_barrier_semaphore()
        pl.semaphore_signal(barrier, inc=1, device_id=peer,
                            device_id_type=pl.DeviceIdType.MESH)
        pl.semaphore_wait(barrier, 1)

        bt_v = bt_ref[:, :]
        j = lax.broadcasted_iota(jnp.int32, (B, NB), 1)
        valid = j < lens_ref[:, :]
        pg = my_y * P_loc + lax.broadcasted_iota(jnp.int32, (B, NB, P_loc), 2)
        hits = jnp.where((bt_v[:, :, None] == pg) & valid[:, :, None], 1.0, 0.0)
        counts = jnp.sum(hits, axis=1)

        kp = lax.broadcasted_iota(jnp.int32, (P_loc, KT), 1) // BS
        pp = lax.broadcasted_iota(jnp.int32, (P_loc, KT), 0)
        E = jnp.where(kp == pp, 1.0, 0.0)
        w = jax.lax.dot_general(counts, E, (((1,), (0,)), ((), ())),
                                preferred_element_type=jnp.float32)

        q = q_ref[:, :, :]
        k = k_ref[:, :, :]
        v = v_ref[:, :, :]
        S = jnp.einsum("bhd,khd->bhk", q, k,
                       preferred_element_type=jnp.float32)
        P = jnp.exp(S * scale) * w[:, None, :]
        l_part = jnp.sum(P, axis=2)
        o_part = jnp.einsum("bhk,khd->bhd", P, v,
                            preferred_element_type=jnp.float32)

        o_send[:, :, :] = o_part
        l_send[:, :] = l_part

        rdma_o = pltpu.make_async_remote_copy(
            src_ref=o_send, dst_ref=o_recv,
            send_sem=send_sems.at[0], recv_sem=recv_sems.at[0],
            device_id=peer, device_id_type=pl.DeviceIdType.MESH)
        rdma_l = pltpu.make_async_remote_copy(
            src_ref=l_send, dst_ref=l_recv,
            send_sem=send_sems.at[1], recv_sem=recv_sems.at[1],
            device_id=peer, device_id_type=pl.DeviceIdType.MESH)
        rdma_o.start()
        rdma_l.start()
        rdma_o.wait()
        rdma_l.wait()

        o_tot = o_part + o_recv[:, :, :]
        l_tot = l_part + l_recv[:, :]
        out_ref[:, :, :] = o_tot / l_tot[:, :, None]

    out = pl.pallas_call(
        body,
        out_shape=jax.ShapeDtypeStruct((B, H, D), jnp.float32),
        in_specs=[pl.BlockSpec(memory_space=pltpu.VMEM)] * 5,
        out_specs=pl.BlockSpec(memory_space=pltpu.VMEM),
        scratch_shapes=[
            pltpu.VMEM((B, H, D), jnp.float32),
            pltpu.VMEM((B, H, D), jnp.float32),
            pltpu.VMEM((B, H), jnp.float32),
            pltpu.VMEM((B, H), jnp.float32),
            pltpu.SemaphoreType.DMA((2,)),
            pltpu.SemaphoreType.DMA((2,)),
        ],
        compiler_params=pltpu.CompilerParams(collective_id=0),
    )(Qs, Kf, Vf, bt, lens2)

    return out.reshape(B, 1, H, D)


# baseline (device time: 61417 ns/iter reference)
import jax
import jax.numpy as jnp
from jax import lax
from jax.experimental import pallas as pl
from jax.experimental.pallas import tpu as pltpu


def kernel(Q, K, V, bt, lens):
    B, _, H, D = Q.shape
    P_loc, BS = K.shape[0], K.shape[1]
    NB = bt.shape[1]
    KT = P_loc * BS
    HD = H * D
    scale = D ** -0.5

    Q2 = Q.reshape(B, HD)
    K2 = K.reshape(KT, HD)
    V2 = V.reshape(KT, HD)
    lens2 = lens.reshape(B, 1)

    def body(q_ref, k_ref, v_ref, bt_ref, lens_ref, out_ref,
             send_buf, recv_buf, send_sem, recv_sem):
        my_x = lax.axis_index("x")
        my_y = lax.axis_index("y")
        peer = (my_x, 1 - my_y)

        barrier = pltpu.get_barrier_semaphore()
        pl.semaphore_signal(barrier, inc=1, device_id=peer,
                            device_id_type=pl.DeviceIdType.MESH)
        pl.semaphore_wait(barrier, 1)

        valid = lax.broadcasted_iota(jnp.int32, (B, NB), 1) < lens_ref[:, :]
        pg = my_y * P_loc + lax.broadcasted_iota(jnp.int32, (P_loc, B, NB), 0)
        hit = (bt_ref[:, :][None, :, :] == pg) & valid[None, :, :]
        counts = jnp.sum(jnp.where(hit, 1.0, 0.0), axis=2)

        kp = lax.broadcasted_iota(jnp.int32, (P_loc, KT), 1) // BS
        pp = lax.broadcasted_iota(jnp.int32, (P_loc, KT), 0)
        E = jnp.where(kp == pp, 1.0, 0.0)
        w = lax.dot_general(counts, E, (((0,), (0,)), ((), ())),
                            preferred_element_type=jnp.float32)

        ones_kd = jnp.ones((KT, D), jnp.float32)
        for h in range(H):
            sl = pl.ds(h * D, D)
            q_h = q_ref[:, sl]
            k_h = k_ref[:, sl]
            v_h = v_ref[:, sl]
            s_h = lax.dot_general(q_h, k_h, (((1,), (1,)), ((), ())),
                                  preferred_element_type=jnp.float32)
            p_h = jnp.exp(s_h * scale) * w
            o_h = lax.dot_general(p_h, v_h, (((1,), (0,)), ((), ())),
                                  preferred_element_type=jnp.float32)
            l_h = lax.dot_general(p_h, ones_kd, (((1,), (0,)), ((), ())),
                                  preferred_element_type=jnp.float32)
            send_buf[:, sl] = o_h
            send_buf[:, pl.ds(HD + h * D, D)] = l_h

        rdma = pltpu.make_async_remote_copy(
            src_ref=send_buf, dst_ref=recv_buf,
            send_sem=send_sem, recv_sem=recv_sem,
            device_id=peer, device_id_type=pl.DeviceIdType.MESH)
        rdma.start()
        rdma.wait()

        o_tot = send_buf[:, :HD] + recv_buf[:, :HD]
        l_tot = send_buf[:, HD:] + recv_buf[:, HD:]
        out_ref[:, :] = o_tot / l_tot

    out = pl.pallas_call(
        body,
        out_shape=jax.ShapeDtypeStruct((B, HD), jnp.float32),
        in_specs=[pl.BlockSpec(memory_space=pltpu.VMEM)] * 5,
        out_specs=pl.BlockSpec(memory_space=pltpu.VMEM),
        scratch_shapes=[
            pltpu.VMEM((B, 2 * HD), jnp.float32),
            pltpu.VMEM((B, 2 * HD), jnp.float32),
            pltpu.SemaphoreType.DMA,
            pltpu.SemaphoreType.DMA,
        ],
        compiler_params=pltpu.CompilerParams(collective_id=0),
    )(Q2, K2, V2, bt, lens2)

    return out.reshape(B, 1, H, D)


# device time: 59300 ns/iter; 1.0357x vs baseline; 1.0357x over previous
import jax
import jax.numpy as jnp
from jax import lax
from jax.experimental import pallas as pl
from jax.experimental.pallas import tpu as pltpu


def kernel(Q, K, V, bt, lens):
    B, _, H, D = Q.shape
    P_loc, BS = K.shape[0], K.shape[1]
    NB = bt.shape[1]
    KT = P_loc * BS
    HD = H * D
    HB = H * B
    scale = D ** -0.5

    Qt = Q.reshape(B, H, D).transpose(1, 0, 2) * scale
    Qbig = jnp.einsum("hbd,hg->hbgd", Qt, jnp.eye(H, dtype=Q.dtype))
    Qbig = Qbig.reshape(HB, HD)
    K2 = K.reshape(KT, HD)
    V2 = V.reshape(KT, HD)
    lens2 = lens.reshape(B, 1)

    def body(q_ref, k_ref, v_ref, bt_ref, lens_ref, out_ref,
             send_buf, recv_buf, send_sem, recv_sem):
        my_x = lax.axis_index("x")
        my_y = lax.axis_index("y")
        peer = (my_x, 1 - my_y)

        barrier = pltpu.get_barrier_semaphore()
        pl.semaphore_signal(barrier, inc=1, device_id=peer,
                            device_id_type=pl.DeviceIdType.MESH)
        pl.semaphore_wait(barrier, 1)

        valid = lax.broadcasted_iota(jnp.int32, (B, NB), 1) < lens_ref[:, :]
        pg = my_y * P_loc + lax.broadcasted_iota(jnp.int32, (P_loc, B, NB), 0)
        hit = (bt_ref[:, :][None, :, :] == pg) & valid[None, :, :]
        counts = jnp.sum(jnp.where(hit, 1.0, 0.0), axis=2)

        kp = lax.broadcasted_iota(jnp.int32, (P_loc, KT), 1) // BS
        pp = lax.broadcasted_iota(jnp.int32, (P_loc, KT), 0)
        E = jnp.where(kp == pp, 1.0, 0.0)
        w = lax.dot_general(counts, E, (((0,), (0,)), ((), ())),
                            preferred_element_type=jnp.float32)
        rb = lax.broadcasted_iota(jnp.int32, (B, HB), 1) % B
        bb = lax.broadcasted_iota(jnp.int32, (B, HB), 0)
        T = jnp.where(rb == bb, 1.0, 0.0)
        wbig = lax.dot_general(T, w, (((0,), (0,)), ((), ())),
                               preferred_element_type=jnp.float32)

        s = lax.dot_general(q_ref[:, :], k_ref[:, :],
                            (((1,), (1,)), ((), ())),
                            preferred_element_type=jnp.float32)
        p = jnp.exp(s) * wbig

        obig = lax.dot_general(p, v_ref[:, :], (((1,), (0,)), ((), ())),
                               preferred_element_type=jnp.float32)
        lbig = lax.dot_general(p, jnp.ones((KT, D), jnp.float32),
                               (((1,), (0,)), ((), ())),
                               preferred_element_type=jnp.float32)

        for h in range(H):
            sl = pl.ds(h * D, D)
            send_buf[:, sl] = obig[h * B:(h + 1) * B, h * D:(h + 1) * D]
            send_buf[:, pl.ds(HD + h * D, D)] = lbig[h * B:(h + 1) * B, :]

        rdma = pltpu.make_async_remote_copy(
            src_ref=send_buf, dst_ref=recv_buf,
            send_sem=send_sem, recv_sem=recv_sem,
            device_id=peer, device_id_type=pl.DeviceIdType.MESH)
        rdma.start()
        rdma.wait()

        o_tot = send_buf[:, :HD] + recv_buf[:, :HD]
        l_tot = send_buf[:, HD:] + recv_buf[:, HD:]
        out_ref[:, :] = o_tot / l_tot

    out = pl.pallas_call(
        body,
        out_shape=jax.ShapeDtypeStruct((B, HD), jnp.float32),
        in_specs=[pl.BlockSpec(memory_space=pltpu.VMEM)] * 5,
        out_specs=pl.BlockSpec(memory_space=pltpu.VMEM),
        scratch_shapes=[
            pltpu.VMEM((B, 2 * HD), jnp.float32),
            pltpu.VMEM((B, 2 * HD), jnp.float32),
            pltpu.SemaphoreType.DMA,
            pltpu.SemaphoreType.DMA,
        ],
        compiler_params=pltpu.CompilerParams(collective_id=0),
    )(Qbig, K2, V2, bt, lens2)

    return out.reshape(B, 1, H, D)
